# baseline (speedup 1.0000x reference)
import functools

import jax
import jax.numpy as jnp
from jax import lax
from jax.experimental import pallas as pl
from jax.experimental.pallas import tpu as pltpu
from jax.experimental.pallas import tpu_sc as plsc

_EMBED = 64
_SEQ = 200
_NC = 2
_NS = 16
_NW = _NC * _NS
_CHUNK = 128
_NBUF = 8
_LEAD = 4
_LANE = 16


def _emb_body(rows_per_worker, num_chunks,
              x_ref, posx_ref, tab_ref, out_ref,
              idx_v, pos_v, buf_v, sem_io, gsem, ssem):
    wid = lax.axis_index("s") * _NC + lax.axis_index("c")
    wbase = pl.multiple_of(wid * rows_per_worker, _CHUNK)

    pltpu.async_copy(x_ref.at[pl.ds(wbase, rows_per_worker)], idx_v, sem_io).wait()
    pltpu.async_copy(posx_ref, pos_v, sem_io).wait()

    def gather(j, b):
        off = pl.multiple_of(j * _CHUNK, _CHUNK)
        idx = idx_v.at[pl.ds(off, _CHUNK)]
        return pltpu.make_async_copy(tab_ref.at[idx], buf_v.at[b], gsem.at[b])

    def scatter(j, b):
        off = pl.multiple_of(wbase + j * _CHUNK, _CHUNK)
        dst = out_ref.at[pl.ds(off, _CHUNK)]
        return pltpu.make_async_copy(buf_v.at[b], dst, ssem.at[b])

    _NVREG = _CHUNK * _EMBED // _LANE

    def add_pos(j, b):
        s = lax.rem(j * _CHUNK, _SEQ)
        pbase = s * _EMBED

        @plsc.parallel_loop(0, _CHUNK, step=2)
        def _(r):
            for u in range(2):
                row = r + u
                poff = pbase + row * _EMBED
                for k in range(_EMBED // _LANE):
                    sl = pl.ds(k * _LANE, _LANE)
                    buf_v[b, row, sl] = (
                        buf_v[b, row, sl]
                        + pos_v[pl.ds(poff + k * _LANE, _LANE)]
                    )

    def do_chunk(j, b, wait_prev, issue_next):
        bn = (b + _LEAD) % _NBUF
        gather(j, b).wait()
        add_pos(j, b)
        scatter(j, b).start()
        if wait_prev:
            scatter(j - _LEAD, bn).wait()
        if issue_next:
            gather(j + _LEAD, bn).start()

    for j in range(_LEAD):
        gather(j, j % _NBUF).start()

    for j in range(_NBUF):
        do_chunk(j, j % _NBUF, wait_prev=(j >= _LEAD), issue_next=True)

    num_groups = num_chunks // _NBUF

    def group_body(g, carry):
        for b in range(_NBUF):
            j = g * _NBUF + b
            do_chunk(j, b, wait_prev=True, issue_next=True)
        return carry

    lax.fori_loop(1, num_groups - 1, group_body, 0)

    for b in range(_NBUF):
        j = num_chunks - _NBUF + b
        do_chunk(j, b, wait_prev=True, issue_next=(b < _NBUF - _LEAD))

    for b in range(_NBUF - _LEAD, _NBUF):
        scatter(num_chunks - _NBUF + b, b).wait()


@functools.partial(jax.jit, static_argnums=())
def _run(x_flat, posx, token_table):
    rows = x_flat.shape[0]
    rows_per_worker = rows // _NW
    num_chunks = rows_per_worker // _CHUNK
    mesh = plsc.VectorSubcoreMesh(core_axis_name="c", subcore_axis_name="s")
    body = functools.partial(_emb_body, rows_per_worker, num_chunks)
    fn = pl.kernel(
        body,
        mesh=mesh,
        out_type=jax.ShapeDtypeStruct((rows, _EMBED), jnp.float32),
        scratch_types=[
            pltpu.VMEM((rows_per_worker,), jnp.int32),
            pltpu.VMEM(((_SEQ + _CHUNK) * _EMBED,), jnp.float32),
            pltpu.VMEM((_NBUF, _CHUNK, _EMBED), jnp.float32),
            pltpu.SemaphoreType.DMA,
            pltpu.SemaphoreType.DMA((_NBUF,)),
            pltpu.SemaphoreType.DMA((_NBUF,)),
        ],
        compiler_params=pltpu.CompilerParams(use_tc_tiling_on_sc=False),
    )
    return fn(x_flat, posx, token_table)


def kernel(x, token_table, pos_table):
    b, l = x.shape
    e = token_table.shape[1]
    x_flat = x.reshape(b * l).astype(jnp.int32)
    posx = jnp.concatenate([pos_table[:l], pos_table[:_CHUNK]], axis=0).reshape(-1)
    out = _run(x_flat, posx, token_table)
    return out.reshape(b, l, e)


# speedup vs baseline: 1.0826x; 1.0826x over previous
import functools

import jax
import jax.numpy as jnp
from jax import lax
from jax.experimental import pallas as pl
from jax.experimental.pallas import tpu as pltpu
from jax.experimental.pallas import tpu_sc as plsc

_EMBED = 64
_PAD = 128
_SEQ = 200
_NC = 2
_NS = 16
_NW = _NC * _NS
_C0 = 128
_C1 = _SEQ - _C0
_LANE = 16


def _emb_body(seqs_per_worker, x_ref, pos_ref, tab_ref, out_ref,
              idx_v, pos_v, gbuf_v, abuf_v, sem_io, gsem, ssem):
    wid = lax.axis_index("s") * _NC + lax.axis_index("c")
    half = seqs_per_worker // 2 * _SEQ
    wbase = wid * seqs_per_worker

    def stage_idx(h):
        off = pl.multiple_of(wbase * _SEQ + h * half, 8)
        pltpu.async_copy(x_ref.at[pl.ds(off, half)], idx_v, sem_io).wait()

    stage_idx(0)
    pltpu.async_copy(pos_ref, pos_v, sem_io).wait()

    def gathers(si, b):
        loc = lax.rem(si, seqs_per_worker // 2)
        base = pl.multiple_of(loc * _SEQ, 8)
        i0 = idx_v.at[pl.ds(base, _C0)]
        i1 = idx_v.at[pl.ds(base + _C0, _C1)]
        c0 = pltpu.make_async_copy(
            tab_ref.at[i0], gbuf_v.at[b, pl.ds(0, _C0)], gsem.at[b])
        c1 = pltpu.make_async_copy(
            tab_ref.at[i1], gbuf_v.at[b, pl.ds(_C0, _C1)], gsem.at[b])
        return c0, c1

    def scatter(si, b):
        bb = wbase + si
        return pltpu.make_async_copy(abuf_v.at[b], out_ref.at[bb], ssem.at[b])

    def add_pos(b):
        @plsc.parallel_loop(0, _SEQ, step=2)
        def _(r):
            for u in range(2):
                row = r + u
                for k in range(_EMBED // _LANE):
                    sl = pl.ds(k * _LANE, _LANE)
                    abuf_v[b, row, sl] = (
                        gbuf_v[b, row, sl]
                        + pos_v[pl.ds(row * _EMBED + k * _LANE, _LANE)]
                    )

    def issue_gathers(si, b):
        c0, c1 = gathers(si, b)
        c0.start()
        c1.start()

    def wait_gathers(si, b):
        c0, c1 = gathers(si, b)
        c0.wait()
        c1.wait()

    def do_seq(si, b, wait_prev_scatter, issue_next):
        wait_gathers(si, b)
        if issue_next:
            issue_gathers(si + 1, 1 - b)
        if wait_prev_scatter:
            scatter(si - 2, b).wait()
        add_pos(b)
        scatter(si, b).start()

    nsw = seqs_per_worker

    def group_body(g, carry):
        si = g * 2
        do_seq(si, 0, wait_prev_scatter=True, issue_next=True)
        do_seq(si + 1, 1, wait_prev_scatter=True, issue_next=True)
        return carry

    issue_gathers(0, 0)
    do_seq(0, 0, wait_prev_scatter=False, issue_next=True)
    do_seq(1, 1, wait_prev_scatter=False, issue_next=True)
    lax.fori_loop(1, nsw // 4 - 1, group_body, 0)
    do_seq(nsw // 2 - 2, 0, wait_prev_scatter=True, issue_next=True)
    do_seq(nsw // 2 - 1, 1, wait_prev_scatter=True, issue_next=False)

    stage_idx(1)
    issue_gathers(nsw // 2, 0)
    do_seq(nsw // 2, 0, wait_prev_scatter=True, issue_next=True)
    do_seq(nsw // 2 + 1, 1, wait_prev_scatter=True, issue_next=True)
    lax.fori_loop(nsw // 4 + 1, nsw // 2 - 1, group_body, 0)
    do_seq(nsw - 2, 0, wait_prev_scatter=True, issue_next=True)
    do_seq(nsw - 1, 1, wait_prev_scatter=True, issue_next=False)
    scatter(nsw - 2, 0).wait()
    scatter(nsw - 1, 1).wait()


@jax.jit
def _run(x_flat, pos_flat, tab_pad):
    rows = x_flat.shape[0]
    nseq = rows // _SEQ
    seqs_per_worker = nseq // _NW
    mesh = plsc.VectorSubcoreMesh(core_axis_name="c", subcore_axis_name="s")
    body = functools.partial(_emb_body, seqs_per_worker)
    fn = pl.kernel(
        body,
        mesh=mesh,
        out_type=jax.ShapeDtypeStruct((nseq, _SEQ, _EMBED), jnp.float32),
        scratch_types=[
            pltpu.VMEM((seqs_per_worker // 2 * _SEQ,), jnp.int32),
            pltpu.VMEM((_SEQ * _EMBED,), jnp.float32),
            pltpu.VMEM((2, _SEQ, _PAD), jnp.float32),
            pltpu.VMEM((2, _SEQ, _EMBED), jnp.float32),
            pltpu.SemaphoreType.DMA,
            pltpu.SemaphoreType.DMA((2,)),
            pltpu.SemaphoreType.DMA((2,)),
        ],
        compiler_params=pltpu.CompilerParams(use_tc_tiling_on_sc=True),
    )
    return fn(x_flat, pos_flat, tab_pad)


def kernel(x, token_table, pos_table):
    b, l = x.shape
    e = token_table.shape[1]
    x_flat = x.reshape(b * l).astype(jnp.int32)
    pos_flat = pos_table[:l].reshape(-1)
    tab_pad = jnp.pad(token_table, ((0, 0), (0, _PAD - e)))
    return _run(x_flat, pos_flat, tab_pad)
